# Initial kernel scaffold; baseline (speedup 1.0000x reference)
import jax
import jax.numpy as jnp
from jax.experimental import pallas as pl


def kernel(x, token_table, pos_table):
    raise NotImplementedError("write your pallas kernel here")


# speedup vs baseline: 3.1094x; 3.1094x over previous
import functools

import jax
import jax.numpy as jnp
from jax import lax
from jax.experimental import pallas as pl
from jax.experimental.pallas import tpu as pltpu
from jax.experimental.pallas import tpu_sc as plsc

NC = 2
NS = 16
NW = NC * NS
LANES = 16


def _make_sc_kernel(B, L, V, D):
    assert B % NW == 0
    per_w = B // NW
    N = B * L
    splits = []
    off = 0
    while off < L:
        n = min((L - off) // 8 * 8, 128) or (L - off)
        splits.append((off, n))
        off += n
    mesh = plsc.VectorSubcoreMesh(core_axis_name="c", subcore_axis_name="s")

    @functools.partial(
        pl.kernel,
        out_type=jax.ShapeDtypeStruct((N, D), jnp.float32),
        mesh=mesh,
        scratch_types=[
            pltpu.VMEM((L,), jnp.int32),
            pltpu.VMEM((L, D), jnp.float32),
            pltpu.VMEM((L, D), jnp.float32),
            pltpu.SemaphoreType.DMA,
        ],
        compiler_params=pltpu.CompilerParams(use_tc_tiling_on_sc=False),
    )
    def k(x_hbm, tok_hbm, pos_hbm, out_hbm, idx_v, tok_v, pos_v, sem):
        cid = lax.axis_index("c")
        sid = lax.axis_index("s")
        wid = sid * NC + cid
        pltpu.sync_copy(pos_hbm, pos_v)

        def chunk_body(g, carry):
            base = (wid * per_w + g) * L
            pltpu.sync_copy(x_hbm.at[pl.ds(base, L)], idx_v)
            cps = [
                pltpu.async_copy(
                    tok_hbm.at[idx_v.at[pl.ds(o, n)]],
                    tok_v.at[pl.ds(o, n)],
                    sem,
                )
                for (o, n) in splits
            ]
            for cp in cps:
                cp.wait()

            def add_body(l, c2):
                for c in range(D // LANES):
                    sl = pl.ds(c * LANES, LANES)
                    tok_v[l, sl] = tok_v[l, sl] + pos_v[l, sl]
                return c2

            lax.fori_loop(0, L, add_body, 0)
            pltpu.sync_copy(tok_v, out_hbm.at[pl.ds(base, L)])
            return carry

        lax.fori_loop(0, per_w, chunk_body, 0)

    return k


def kernel(x, token_table, pos_table):
    B, L = x.shape
    V, D = token_table.shape
    k = _make_sc_kernel(B, L, V, D)
    x_flat = x.reshape(B * L).astype(jnp.int32)
    out = k(x_flat, token_table, pos_table)
    return out.reshape(B, L, D)


# speedup vs baseline: 3.4631x; 1.1138x over previous
import functools

import jax
import jax.numpy as jnp
from jax import lax
from jax.experimental import pallas as pl
from jax.experimental.pallas import tpu as pltpu
from jax.experimental.pallas import tpu_sc as plsc

NC = 2
NS = 16
NW = NC * NS
LANES = 16
ROWS_PER_CHUNK = 2


def _make_sc_kernel(B, L, V, D):
    assert B % (NW * ROWS_PER_CHUNK) == 0
    assert L % 2 == 0 and D == 64
    per_w = B // NW
    n_chunks = per_w // ROWS_PER_CHUNK
    C = ROWS_PER_CHUNK * L
    CP = C // 2
    HL = L // 2
    N = B * L
    splits = []
    off = 0
    while off < C:
        n = min(C - off, 128)
        splits.append((off, n))
        off += n
    mesh = plsc.VectorSubcoreMesh(core_axis_name="c", subcore_axis_name="s")

    @functools.partial(
        pl.kernel,
        out_type=jax.ShapeDtypeStruct((N // 2, 2 * D), jnp.float32),
        mesh=mesh,
        scratch_types=[
            pltpu.VMEM((C,), jnp.int32),
            pltpu.VMEM((C, D), jnp.float32),
            pltpu.VMEM((CP, 2 * D), jnp.float32),
            pltpu.VMEM((HL, 2 * D), jnp.float32),
            pltpu.SemaphoreType.DMA,
        ],
        compiler_params=pltpu.CompilerParams(use_tc_tiling_on_sc=False),
    )
    def k(x_hbm, tok_hbm, pos_hbm, out_hbm, idx_v, tok_v, out_v, pos_v, sem):
        cid = lax.axis_index("c")
        sid = lax.axis_index("s")
        wid = sid * NC + cid
        pltpu.sync_copy(pos_hbm, pos_v)

        def chunk_body(g, carry):
            base = pl.multiple_of((wid * n_chunks + g) * C, C)
            pltpu.sync_copy(x_hbm.at[pl.ds(base, C)], idx_v)
            cps = [
                pltpu.async_copy(
                    tok_hbm.at[idx_v.at[pl.ds(o, n)]],
                    tok_v.at[pl.ds(o, n)],
                    sem,
                )
                for (o, n) in splits
            ]
            for cp in cps:
                cp.wait()

            def add_body(p, c2):
                pos_regs = [
                    pos_v[p, pl.ds(i * LANES, LANES)]
                    for i in range(2 * D // LANES)
                ]
                for rep in range(CP // HL):
                    r = p + rep * HL
                    for h in range(2):
                        for c in range(D // LANES):
                            i = h * (D // LANES) + c
                            sl_out = pl.ds(i * LANES, LANES)
                            sl_tok = pl.ds(c * LANES, LANES)
                            out_v[r, sl_out] = (
                                tok_v[2 * r + h, sl_tok] + pos_regs[i]
                            )
                return c2

            lax.fori_loop(0, HL, add_body, 0)
            pltpu.sync_copy(
                out_v, out_hbm.at[pl.ds(pl.multiple_of(base // 2, CP), CP)]
            )
            return carry

        lax.fori_loop(0, n_chunks, chunk_body, 0)

    return k


def kernel(x, token_table, pos_table):
    B, L = x.shape
    V, D = token_table.shape
    k = _make_sc_kernel(B, L, V, D)
    x_flat = x.reshape(B * L).astype(jnp.int32)
    pos_pairs = pos_table.reshape(L // 2, 2 * D)
    out = k(x_flat, token_table, pos_pairs)
    return out.reshape(B, L, D)
